# baseline (device time: 76520 ns/iter reference)
import os

import jax
import jax.numpy as jnp
from jax import lax
from jax.experimental import pallas as pl
from jax.experimental.pallas import tpu as pltpu

COMM = os.environ.get("NO_COMM") != "1"

NZ = 4
P = 560
N_COL = 1024
N_ROWS = 2048


def _a2av(x, dest_col):
    n, ncol = x.shape

    def body(x_ref, dest_ref, out_ref,
             send_ref, recv_ref, xbf_ref, cnt_send_ref, cnt_recv_ref,
             data_send_sems, data_recv_sems, cnt_send_sems, cnt_recv_sems):
        my_x = lax.axis_index("x")
        my_y = lax.axis_index("y")
        my_z = lax.axis_index("z")

        if COMM:
            barrier_sem = pltpu.get_barrier_semaphore()
            for d in range(1, NZ):
                pl.semaphore_signal(
                    barrier_sem, inc=1,
                    device_id=(my_x, my_y, (my_z + d) % NZ),
                    device_id_type=pl.DeviceIdType.MESH,
                )
            pl.semaphore_wait(barrier_sem, NZ - 1)

        dest_c = dest_ref[...]
        mask = dest_c == lax.broadcasted_iota(jnp.int32, (n, 128), 1)
        counts_row = jnp.sum(mask.astype(jnp.float32), axis=0, keepdims=True
                             ).astype(jnp.int32)
        cnt_send_ref[...] = jnp.where(
            lax.broadcasted_iota(jnp.int32, (8, 128), 0) == 0, counts_row, 0)
        cnt_rdmas = []
        for d in range(1, NZ):
            peer = (my_z + d) % NZ
            if COMM:
                cnt_rdma = pltpu.make_async_remote_copy(
                    src_ref=cnt_send_ref,
                    dst_ref=cnt_recv_ref.at[d - 1],
                    send_sem=cnt_send_sems.at[d - 1],
                    recv_sem=cnt_recv_sems.at[d - 1],
                    device_id=(my_x, my_y, peer),
                    device_id_type=pl.DeviceIdType.MESH,
                )
                cnt_rdma.start()
                cnt_rdmas.append(cnt_rdma)
            else:
                cnt_recv_ref[d - 1] = cnt_send_ref[...]

        NB, B = 8, n // 8
        tri_b = (lax.broadcasted_iota(jnp.int32, (B, B), 1)
                 < lax.broadcasted_iota(jnp.int32, (B, B), 0)
                 ).astype(jnp.bfloat16)
        base = jnp.zeros((1, 128), jnp.float32)
        rank_blocks = []
        for blk in range(NB):
            mask_blk = mask[blk * B:(blk + 1) * B, :]
            cum_blk = jnp.dot(tri_b, mask_blk.astype(jnp.bfloat16),
                              preferred_element_type=jnp.float32) + base
            rank_blocks.append(
                jnp.sum(jnp.where(mask_blk, cum_blk, 0.0),
                        axis=1, keepdims=True))
            base = base + jnp.sum(mask_blk.astype(jnp.float32),
                                  axis=0, keepdims=True)
        rank_c = jnp.concatenate(rank_blocks, axis=0).astype(jnp.int32)
        key_c = dest_c * P + rank_c
        iota_blk = lax.broadcasted_iota(jnp.int32, (n, P), 1)
        xbf_ref[...] = x_ref[...].astype(jnp.bfloat16)
        H = ncol // 2

        def pack_sel(t):
            return (key_c == t * P + iota_blk).astype(jnp.bfloat16)

        def pack_half(sel, h):
            chunk = lax.dot_general(
                sel, xbf_ref[:, h * H:(h + 1) * H],
                dimension_numbers=(((0,), (0,)), ((), ())),
                preferred_element_type=jnp.float32)
            return chunk.astype(jnp.bfloat16)

        data_rdmas = {}
        for d in range(NZ - 1, 0, -1):
            peer = (my_z + d) % NZ
            sel = pack_sel(peer)
            for h in range(2):
                send_ref[d - 1, :, h * H:(h + 1) * H] = pack_half(sel, h)
                if COMM:
                    data_rdma = pltpu.make_async_remote_copy(
                        src_ref=send_ref.at[d - 1, :, pl.ds(h * H, H)],
                        dst_ref=recv_ref.at[d, :, pl.ds(h * H, H)],
                        send_sem=data_send_sems.at[d - 1, h],
                        recv_sem=data_recv_sems.at[d - 1, h],
                        device_id=(my_x, my_y, peer),
                        device_id_type=pl.DeviceIdType.MESH,
                    )
                    data_rdma.start()
                    data_rdmas[d, h] = data_rdma
            if not COMM:
                recv_ref[d] = send_ref[d - 1]

        sel_own = pack_sel(my_z)
        for h in range(2):
            recv_ref[0, :, h * H:(h + 1) * H] = pack_half(sel_own, h)

        for cnt_rdma in cnt_rdmas:
            cnt_rdma.wait()
        col_mask = lax.broadcasted_iota(jnp.int32, (8, 128), 1) == my_z
        row_mask = lax.broadcasted_iota(jnp.int32, (8, 128), 0) == 0
        lmask = col_mask & row_mask

        def _len_of(plane):
            return jnp.sum(jnp.where(lmask, plane, 0))

        l_by_d = [_len_of(cnt_send_ref[...])] + [
            _len_of(cnt_recv_ref[d - 1]) for d in range(1, NZ)
        ]

        len_src, slot_src = [], []
        for s in range(NZ):
            d_s = (my_z - s) % NZ
            ln = l_by_d[0]
            for d in range(1, NZ):
                ln = jnp.where(d_s == d, l_by_d[d], ln)
            len_src.append(ln)
            slot_src.append(d_s)
        starts = [jnp.int32(0)]
        for s in range(1, NZ):
            starts.append(starts[s - 1] + len_src[s - 1])

        j2 = lax.broadcasted_iota(jnp.int32, (n, 1), 0)
        s_idx = jnp.zeros((n, 1), jnp.int32)
        for s in range(1, NZ):
            s_idx = s_idx + (j2 >= starts[s]).astype(jnp.int32)
        start_j = jnp.full((n, 1), starts[0], jnp.int32)
        slot_j = jnp.full((n, 1), slot_src[0], jnp.int32)
        for s in range(1, NZ):
            sel_s = s_idx == s
            start_j = jnp.where(sel_s, starts[s], start_j)
            slot_j = jnp.where(sel_s, slot_src[s], slot_j)
        col_j = slot_j * P + (j2 - start_j)

        def partial(gsel, m, h):
            return jnp.dot(gsel, recv_ref[m, :, h * H:(h + 1) * H],
                           preferred_element_type=jnp.float32
                           ).astype(jnp.bfloat16)

        gsel0 = (col_j == iota_blk).astype(jnp.bfloat16)
        for h in range(2):
            out_ref[:, h * H:(h + 1) * H] = partial(gsel0, 0, h)
        for d in range(1, NZ):
            gsel = (col_j == d * P + iota_blk).astype(jnp.bfloat16)
            for h in range(2):
                if COMM:
                    data_rdmas[d, h].wait()
                out_ref[:, h * H:(h + 1) * H] = (
                    out_ref[:, h * H:(h + 1) * H] + partial(gsel, d, h))

    return pl.pallas_call(
        body,
        out_shape=jax.ShapeDtypeStruct((n, ncol), jnp.bfloat16),
        in_specs=[
            pl.BlockSpec(memory_space=pltpu.VMEM),
            pl.BlockSpec(memory_space=pltpu.VMEM),
        ],
        out_specs=pl.BlockSpec(memory_space=pltpu.VMEM),
        scratch_shapes=[
            pltpu.VMEM((NZ - 1, P, N_COL), jnp.bfloat16),
            pltpu.VMEM((NZ, P, N_COL), jnp.bfloat16),
            pltpu.VMEM((N_ROWS, N_COL), jnp.bfloat16),
            pltpu.VMEM((8, 128), jnp.int32),
            pltpu.VMEM((NZ - 1, 8, 128), jnp.int32),
            pltpu.SemaphoreType.DMA((NZ - 1, 2)),
            pltpu.SemaphoreType.DMA((NZ - 1, 2)),
            pltpu.SemaphoreType.DMA((NZ - 1,)),
            pltpu.SemaphoreType.DMA((NZ - 1,)),
        ],
        compiler_params=pltpu.CompilerParams(
            collective_id=0 if COMM else None,
            vmem_limit_bytes=56 * 1024 * 1024),
    )(x, dest_col)


def kernel(x, dest):
    n, _ = x.shape
    return _a2av(x, dest.astype(jnp.int32).reshape(n, 1))


# device time: 63321 ns/iter; 1.2084x vs baseline; 1.2084x over previous
import os

import jax
import jax.numpy as jnp
from jax import lax
from jax.experimental import pallas as pl
from jax.experimental.pallas import tpu as pltpu

COMM = os.environ.get("NO_COMM") != "1"

NZ = 4
P = 560
N_COL = 1024
N_ROWS = 2048
H = N_COL // 2


def _a2av(x, dest_col):
    n, ncol = x.shape

    def body(x_ref, dest_ref, out_ref,
             send_ref, recv_ref, xbf_ref, stage_ref, cnt_send_ref,
             cnt_recv_ref, z_send_sems, z_recv_sems, xf_send_sems,
             xf_recv_sems, cnt_send_sems, cnt_recv_sems):
        my_x = lax.axis_index("x")
        my_y = lax.axis_index("y")
        my_z = lax.axis_index("z")
        hx = my_x
        px = 1 - my_x

        if COMM:
            barrier_sem = pltpu.get_barrier_semaphore()
            for d in range(1, NZ):
                pl.semaphore_signal(
                    barrier_sem, inc=1,
                    device_id=(my_x, my_y, (my_z + d) % NZ),
                    device_id_type=pl.DeviceIdType.MESH,
                )
            pl.semaphore_signal(
                barrier_sem, inc=1,
                device_id=(px, my_y, my_z),
                device_id_type=pl.DeviceIdType.MESH,
            )
            pl.semaphore_wait(barrier_sem, NZ)

        dest_c = dest_ref[...]
        mask = dest_c == lax.broadcasted_iota(jnp.int32, (n, 128), 1)
        counts_row = jnp.sum(mask.astype(jnp.float32), axis=0, keepdims=True
                             ).astype(jnp.int32)
        cnt_send_ref[...] = jnp.where(
            lax.broadcasted_iota(jnp.int32, (8, 128), 0) == 0, counts_row, 0)
        cnt_rdmas = []
        for d in range(1, NZ):
            peer = (my_z + d) % NZ
            if COMM:
                cnt_rdma = pltpu.make_async_remote_copy(
                    src_ref=cnt_send_ref,
                    dst_ref=cnt_recv_ref.at[d - 1],
                    send_sem=cnt_send_sems.at[d - 1],
                    recv_sem=cnt_recv_sems.at[d - 1],
                    device_id=(my_x, my_y, peer),
                    device_id_type=pl.DeviceIdType.MESH,
                )
                cnt_rdma.start()
                cnt_rdmas.append(cnt_rdma)
            else:
                cnt_recv_ref[d - 1] = cnt_send_ref[...]

        NB, B = 8, n // 8
        tri_b = (lax.broadcasted_iota(jnp.int32, (B, B), 1)
                 < lax.broadcasted_iota(jnp.int32, (B, B), 0)
                 ).astype(jnp.bfloat16)
        base = jnp.zeros((1, 128), jnp.float32)
        rank_blocks = []
        for blk in range(NB):
            mask_blk = mask[blk * B:(blk + 1) * B, :]
            cum_blk = jnp.dot(tri_b, mask_blk.astype(jnp.bfloat16),
                              preferred_element_type=jnp.float32) + base
            rank_blocks.append(
                jnp.sum(jnp.where(mask_blk, cum_blk, 0.0),
                        axis=1, keepdims=True))
            base = base + jnp.sum(mask_blk.astype(jnp.float32),
                                  axis=0, keepdims=True)
        rank_c = jnp.concatenate(rank_blocks, axis=0).astype(jnp.int32)
        key_c = dest_c * P + rank_c
        iota_blk = lax.broadcasted_iota(jnp.int32, (n, P), 1)
        xbf_ref[0] = x_ref[:, 0:H].astype(jnp.bfloat16)
        xbf_ref[1] = x_ref[:, H:ncol].astype(jnp.bfloat16)

        def pack_sel(t):
            return (key_c == t * P + iota_blk).astype(jnp.bfloat16)

        def pack_half(sel, h):
            chunk = lax.dot_general(
                sel, xbf_ref[h],
                dimension_numbers=(((0,), (0,)), ((), ())),
                preferred_element_type=jnp.float32)
            return chunk.astype(jnp.bfloat16)

        z_rdmas = {}
        for d in range(NZ - 1, 0, -1):
            peer = (my_z + d) % NZ
            sel = pack_sel(peer)
            if COMM:
                send_ref[d - 1] = pack_half(sel, hx)
                z_rdma = pltpu.make_async_remote_copy(
                    src_ref=send_ref.at[d - 1],
                    dst_ref=recv_ref.at[d, hx],
                    send_sem=z_send_sems.at[d - 1],
                    recv_sem=z_recv_sems.at[d - 1],
                    device_id=(my_x, my_y, peer),
                    device_id_type=pl.DeviceIdType.MESH,
                )
                z_rdma.start()
                z_rdmas[d] = z_rdma
            else:
                for h in range(2):
                    recv_ref[d, h] = pack_half(sel, h)

        sel_own = pack_sel(my_z)
        for h in range(2):
            recv_ref[0, h] = pack_half(sel_own, h)

        for cnt_rdma in cnt_rdmas:
            cnt_rdma.wait()
        col_mask = lax.broadcasted_iota(jnp.int32, (8, 128), 1) == my_z
        row_mask = lax.broadcasted_iota(jnp.int32, (8, 128), 0) == 0
        lmask = col_mask & row_mask

        def _len_of(plane):
            return jnp.sum(jnp.where(lmask, plane, 0))

        l_by_d = [_len_of(cnt_send_ref[...])] + [
            _len_of(cnt_recv_ref[d - 1]) for d in range(1, NZ)
        ]

        len_src, slot_src = [], []
        for s in range(NZ):
            d_s = (my_z - s) % NZ
            ln = l_by_d[0]
            for d in range(1, NZ):
                ln = jnp.where(d_s == d, l_by_d[d], ln)
            len_src.append(ln)
            slot_src.append(d_s)
        starts = [jnp.int32(0)]
        for s in range(1, NZ):
            starts.append(starts[s - 1] + len_src[s - 1])

        j2 = lax.broadcasted_iota(jnp.int32, (n, 1), 0)
        s_idx = jnp.zeros((n, 1), jnp.int32)
        for s in range(1, NZ):
            s_idx = s_idx + (j2 >= starts[s]).astype(jnp.int32)
        start_j = jnp.full((n, 1), starts[0], jnp.int32)
        slot_j = jnp.full((n, 1), slot_src[0], jnp.int32)
        for s in range(1, NZ):
            sel_s = s_idx == s
            start_j = jnp.where(sel_s, starts[s], start_j)
            slot_j = jnp.where(sel_s, slot_src[s], slot_j)
        col_j = slot_j * P + (j2 - start_j)

        def partial(gsel, m, h):
            return jnp.dot(gsel, recv_ref[m, h],
                           preferred_element_type=jnp.float32
                           ).astype(jnp.bfloat16)

        gsel0 = (col_j == iota_blk).astype(jnp.bfloat16)
        for h in range(2):
            stage_ref[h] = partial(gsel0, 0, h)

        gsels = {d: (col_j == d * P + iota_blk).astype(jnp.bfloat16)
                 for d in range(1, NZ)}
        xf_rdmas = {}
        for d in range(1, NZ):
            if COMM:
                z_rdmas[d].wait()
                xf = pltpu.make_async_remote_copy(
                    src_ref=recv_ref.at[d, hx],
                    dst_ref=recv_ref.at[d, hx],
                    send_sem=xf_send_sems.at[d - 1],
                    recv_sem=xf_recv_sems.at[d - 1],
                    device_id=(px, my_y, my_z),
                    device_id_type=pl.DeviceIdType.MESH,
                )
                xf.start()
                xf_rdmas[d] = xf
                stage_ref[hx] = stage_ref[hx] + partial(gsels[d], d, hx)
            else:
                for h in range(2):
                    stage_ref[h] = stage_ref[h] + partial(gsels[d], d, h)
        if COMM:
            for d in range(1, NZ):
                xf_rdmas[d].wait()
                stage_ref[px] = stage_ref[px] + partial(gsels[d], d, px)

        out_ref[:, 0:H] = stage_ref[0]
        out_ref[:, H:ncol] = stage_ref[1]

    return pl.pallas_call(
        body,
        out_shape=jax.ShapeDtypeStruct((n, ncol), jnp.bfloat16),
        in_specs=[
            pl.BlockSpec(memory_space=pltpu.VMEM),
            pl.BlockSpec(memory_space=pltpu.VMEM),
        ],
        out_specs=pl.BlockSpec(memory_space=pltpu.VMEM),
        scratch_shapes=[
            pltpu.VMEM((NZ - 1, P, H), jnp.bfloat16),
            pltpu.VMEM((NZ, 2, P, H), jnp.bfloat16),
            pltpu.VMEM((2, N_ROWS, H), jnp.bfloat16),
            pltpu.VMEM((2, N_ROWS, H), jnp.bfloat16),
            pltpu.VMEM((8, 128), jnp.int32),
            pltpu.VMEM((NZ - 1, 8, 128), jnp.int32),
            pltpu.SemaphoreType.DMA((NZ - 1,)),
            pltpu.SemaphoreType.DMA((NZ - 1,)),
            pltpu.SemaphoreType.DMA((NZ - 1,)),
            pltpu.SemaphoreType.DMA((NZ - 1,)),
            pltpu.SemaphoreType.DMA((NZ - 1,)),
            pltpu.SemaphoreType.DMA((NZ - 1,)),
        ],
        compiler_params=pltpu.CompilerParams(
            collective_id=0 if COMM else None,
            vmem_limit_bytes=56 * 1024 * 1024),
    )(x, dest_col)


def kernel(x, dest):
    n, _ = x.shape
    return _a2av(x, dest.astype(jnp.int32).reshape(n, 1))
